# baseline (speedup 1.0000x reference)
import jax
import jax.numpy as jnp
from jax.experimental import pallas as pl
from jax.experimental.pallas import tpu as pltpu

B, H, M, D, S = 16, 16, 2048, 128, 16
BH = B * H


def _body(pos_ref, kval_ref, vval_ref, kcache_ref, vcache_ref,
          kout_ref, vout_ref):
    kout_ref[...] = kcache_ref[...]
    vout_ref[...] = vcache_ref[...]
    for s in range(S):
        p = pos_ref[s]
        kout_ref[0, pl.ds(p, 1), :] = kval_ref[0, pl.ds(s, 1), :]
        vout_ref[0, pl.ds(p, 1), :] = vval_ref[0, pl.ds(s, 1), :]


def kernel(input_pos, k_val, v_val, k_cache, v_cache):
    kv = k_val.reshape(BH, S, D)
    vv = v_val.reshape(BH, S, D)
    kc = k_cache.reshape(BH, M, D)
    vc = v_cache.reshape(BH, M, D)
    pos = input_pos.astype(jnp.int32)

    out_shape = jax.ShapeDtypeStruct((BH, M, D), jnp.float32)
    val_spec = pl.BlockSpec((1, S, D), lambda i: (i, 0, 0))
    cache_spec = pl.BlockSpec((1, M, D), lambda i: (i, 0, 0))

    k_out, v_out = pl.pallas_call(
        _body,
        grid=(BH,),
        in_specs=[
            pl.BlockSpec(memory_space=pltpu.SMEM),
            val_spec, val_spec, cache_spec, cache_spec,
        ],
        out_specs=[cache_spec, cache_spec],
        out_shape=[out_shape, out_shape],
    )(pos, kv, vv, kc, vc)

    return (k_out.reshape(B, H, M, D), v_out.reshape(B, H, M, D))
